# baseline (speedup 1.0000x reference)
import functools

import jax
import jax.numpy as jnp
from jax import lax
from jax.experimental import pallas as pl
from jax.experimental.pallas import tpu as pltpu
from jax.experimental.pallas import tpu_sc as plsc

_NC = 2
_NS = 16
_NW = _NC * _NS
_L = 16
_NBUF = 2


@functools.lru_cache(maxsize=None)
def _build_sc_add(B, C, R, T):
    b_per_w = B // _NW
    n_chunks = b_per_w
    n_groups = n_chunks // _NBUF
    mesh = plsc.VectorSubcoreMesh(core_axis_name="core", subcore_axis_name="sub")

    @functools.partial(
        pl.kernel,
        mesh=mesh,
        out_type=jax.ShapeDtypeStruct((B, C, R, 128), jnp.float32),
        compiler_params=pltpu.CompilerParams(use_tc_tiling_on_sc=True),
        scratch_types=(
            [pltpu.VMEM((n_chunks, 1), jnp.int32)]
            + [pltpu.VMEM((1, C, R, 128), jnp.float32) for _ in range(_NBUF)]
            + [pltpu.VMEM((1, R, 128), jnp.float32) for _ in range(_NBUF)]
            + [pltpu.VMEM((1, C, R, 128), jnp.float32) for _ in range(_NBUF)]
            + [pltpu.SemaphoreType.DMA for _ in range(2 * _NBUF)]
        ),
    )
    def sc_add(x_hbm, t2_hbm, emb_hbm, out_hbm,
               idx_v, xb0, xb1, eb0, eb1, ob0, ob1,
               si0, si1, so0, so1):
        xb, eb, ob = (xb0, xb1), (eb0, eb1), (ob0, ob1)
        semi, semo = (si0, si1), (so0, so1)
        wid = lax.axis_index("sub") * _NC + lax.axis_index("core")
        base = wid * b_per_w
        pltpu.sync_copy(t2_hbm.at[pl.ds(base, n_chunks)], idx_v)

        def in_descs(j, s):
            row = pl.ds(base + j, 1)
            return (
                pltpu.make_async_copy(x_hbm.at[row], xb[s], semi[s]),
                pltpu.make_async_copy(emb_hbm.at[idx_v.at[j]], eb[s], semi[s]),
            )

        def out_desc(j, s):
            return pltpu.make_async_copy(ob[s], out_hbm.at[pl.ds(base + j, 1)],
                                         semo[s])

        for s in range(_NBUF):
            for dsc in in_descs(s, s):
                dsc.start()

        def group(g, carry):
            for s in range(_NBUF):
                j = g * _NBUF + s
                for dsc in in_descs(j, s):
                    dsc.wait()

                @pl.when(g > 0)
                def _wait_out():
                    out_desc(j - _NBUF, s).wait()

                def inner(r, c2):
                    for q in range(128 // _L):
                        off = pl.ds(q * _L, _L)
                        e = eb[s][0, r, off]
                        for ci in range(C):
                            ob[s][0, ci, r, off] = xb[s][0, ci, r, off] + e
                    return c2

                lax.fori_loop(0, R, inner, 0, unroll=2)

                @pl.when(j + _NBUF < n_chunks)
                def _refill():
                    for dsc in in_descs(j + _NBUF, s):
                        dsc.start()

                out_desc(j, s).start()
            return carry

        lax.fori_loop(0, n_groups, group, 0)
        for s in range(_NBUF):
            out_desc(n_chunks - _NBUF + s, s).wait()

    return sc_add


def kernel(x, t, embedding):
    B, C, d1, d2 = x.shape
    T = embedding.shape[0]
    R = d1 * d2 // 128
    x4 = x.reshape(B, C, R, 128)
    emb3 = embedding.reshape(T, R, 128)
    t2 = t.reshape(B, 1)
    out = _build_sc_add(B, C, R, T)(x4, t2, emb3)
    return out.reshape(B, C, d1, d2)
